# baseline (speedup 1.0000x reference)
import functools

import jax
import jax.numpy as jnp
from jax import lax
from jax.experimental import pallas as pl
from jax.experimental.pallas import tpu as pltpu
from jax.experimental.pallas import tpu_sc as plsc

HIDDEN = 64
NUM_CORES = 2
NUM_SUBCORES = 16
NUM_WORKERS = NUM_CORES * NUM_SUBCORES
SPLIT = 128
NBUF = 4


def _gather_kernel(rows_per_w, seq, idx_hbm, table_hbm, out_hbm,
                   idx_v, rows_v, g0sems, g1sems, wsems):
  n_rounds = rows_per_w // NBUF
  wid = lax.axis_index("s") * NUM_CORES + lax.axis_index("c")
  wbase = wid * rows_per_w
  tail = seq - SPLIT

  pltpu.sync_copy(
      idx_hbm.at[pl.ds(pl.multiple_of(wbase, 8), rows_per_w)], idx_v)

  def gather_copies(j, s):
    return (
        pltpu.make_async_copy(
            table_hbm.at[idx_v.at[j, pl.ds(0, SPLIT)]],
            rows_v.at[s, pl.ds(0, SPLIT)],
            g0sems[s]),
        pltpu.make_async_copy(
            table_hbm.at[idx_v.at[j, pl.ds(SPLIT, tail)]],
            rows_v.at[s, pl.ds(SPLIT, tail)],
            g1sems[s]),
    )

  def start_gather(j, s):
    c0, c1 = gather_copies(j, s)
    c0.start()
    c1.start()

  def wait_gather(j, s):
    c0, c1 = gather_copies(j, s)
    c0.wait()
    c1.wait()

  def write_copy(j, s):
    return pltpu.make_async_copy(
        rows_v.at[s],
        out_hbm.at[pl.ds(pl.multiple_of((wbase + j) * seq, 8), seq)],
        wsems[s])

  for s in range(NBUF):
    start_gather(s, s)

  def body(r, carry):
    for s in range(NBUF):
      j = r * NBUF + s
      wait_gather(j, s)
      write_copy(j, s).start()
    for s in range(NBUF):
      j = r * NBUF + s
      write_copy(j, s).wait()
      start_gather(j + NBUF, s)
    return carry

  lax.fori_loop(0, n_rounds - 1, body, 0)

  r = n_rounds - 1
  for s in range(NBUF):
    j = r * NBUF + s
    wait_gather(j, s)
    write_copy(j, s).start()
  for s in range(NBUF):
    write_copy(r * NBUF + s, s).wait()


def _build_call(n_rows, seq):
  assert n_rows % (NUM_WORKERS * NBUF) == 0
  rows_per_w = n_rows // NUM_WORKERS
  assert rows_per_w % 8 == 0 and seq % 8 == 0 and seq - SPLIT <= 128
  mesh = plsc.VectorSubcoreMesh(core_axis_name="c", subcore_axis_name="s")
  return pl.kernel(
      functools.partial(_gather_kernel, rows_per_w, seq),
      out_type=jax.ShapeDtypeStruct((n_rows * seq, HIDDEN), jnp.float32),
      mesh=mesh,
      scratch_types=[
          pltpu.VMEM((rows_per_w, seq), jnp.int32),
          pltpu.VMEM((NBUF, seq, HIDDEN), jnp.float32),
          [pltpu.SemaphoreType.DMA] * NBUF,
          [pltpu.SemaphoreType.DMA] * NBUF,
          [pltpu.SemaphoreType.DMA] * NBUF,
      ],
      compiler_params=pltpu.CompilerParams(use_tc_tiling_on_sc=False),
  )


@jax.jit
def kernel(input_ids, table):
  n_rows, seq = input_ids.shape
  idx = input_ids.astype(jnp.int32)
  out = _build_call(n_rows, seq)(idx, table)
  return out.reshape(n_rows, seq, HIDDEN)


# speedup vs baseline: 1.0013x; 1.0013x over previous
import functools

import jax
import jax.numpy as jnp
from jax import lax
from jax.experimental import pallas as pl
from jax.experimental.pallas import tpu as pltpu
from jax.experimental.pallas import tpu_sc as plsc

HIDDEN = 64
NUM_CORES = 2
NUM_SUBCORES = 16
NUM_WORKERS = NUM_CORES * NUM_SUBCORES
SPLIT = 128
NBUF = 4


def _gather_kernel(rows_per_w, seq, idx_hbm, table_hbm, out_hbm,
                   idx_v, rows_v, g0sems, g1sems, wsems):
  n_rounds = rows_per_w // NBUF
  wid = lax.axis_index("s") * NUM_CORES + lax.axis_index("c")
  wbase = wid * rows_per_w
  tail = seq - SPLIT

  pltpu.sync_copy(
      idx_hbm.at[pl.ds(pl.multiple_of(wbase, 8), rows_per_w)], idx_v)

  def gather_copies(j, s):
    return (
        pltpu.make_async_copy(
            table_hbm.at[idx_v.at[j, pl.ds(0, SPLIT)]],
            rows_v.at[s, pl.ds(0, SPLIT)],
            g0sems[s]),
        pltpu.make_async_copy(
            table_hbm.at[idx_v.at[j, pl.ds(SPLIT, tail)]],
            rows_v.at[s, pl.ds(SPLIT, tail)],
            g1sems[s]),
    )

  def start_gather(j, s):
    c0, c1 = gather_copies(j, s)
    c0.start()
    c1.start()

  def wait_gather(j, s):
    c0, c1 = gather_copies(j, s)
    c0.wait()
    c1.wait()

  def write_copy(j, s):
    return pltpu.make_async_copy(
        rows_v.at[s],
        out_hbm.at[wbase + j],
        wsems[s])

  for s in range(NBUF):
    start_gather(s, s)

  def body(r, carry):
    for s in range(NBUF):
      j = r * NBUF + s
      wait_gather(j, s)
      write_copy(j, s).start()
    for s in range(NBUF):
      j = r * NBUF + s
      write_copy(j, s).wait()
      start_gather(j + NBUF, s)
    return carry

  lax.fori_loop(0, n_rounds - 1, body, 0)

  r = n_rounds - 1
  for s in range(NBUF):
    j = r * NBUF + s
    wait_gather(j, s)
    write_copy(j, s).start()
  for s in range(NBUF):
    write_copy(r * NBUF + s, s).wait()


def _build_call(n_rows, seq):
  assert n_rows % (NUM_WORKERS * NBUF) == 0
  rows_per_w = n_rows // NUM_WORKERS
  assert rows_per_w % 8 == 0 and seq % 8 == 0 and seq - SPLIT <= 128
  mesh = plsc.VectorSubcoreMesh(core_axis_name="c", subcore_axis_name="s")
  return pl.kernel(
      functools.partial(_gather_kernel, rows_per_w, seq),
      out_type=jax.ShapeDtypeStruct((n_rows, seq, HIDDEN), jnp.float32),
      mesh=mesh,
      scratch_types=[
          pltpu.VMEM((rows_per_w, seq), jnp.int32),
          pltpu.VMEM((NBUF, seq, HIDDEN), jnp.float32),
          [pltpu.SemaphoreType.DMA] * NBUF,
          [pltpu.SemaphoreType.DMA] * NBUF,
          [pltpu.SemaphoreType.DMA] * NBUF,
      ],
      compiler_params=pltpu.CompilerParams(use_tc_tiling_on_sc=False),
  )


@jax.jit
def kernel(input_ids, table):
  n_rows, seq = input_ids.shape
  idx = input_ids.astype(jnp.int32)
  return _build_call(n_rows, seq)(idx, table)
